# baseline (speedup 1.0000x reference)
import jax
import jax.numpy as jnp
from jax.experimental import pallas as pl

_BLOCK = 16640


def _body(x_ref, c_ref, o_ref):
    xb = x_ref[...]
    xt = xb.T
    x2 = jnp.sum(xt * xt, axis=0, keepdims=True)
    c = c_ref[...]
    c2 = jnp.sum(c * c, axis=1, keepdims=True)
    m = jax.lax.dot_general(c, xt, (((1,), (0,)), ((), ())),
                            preferred_element_type=jnp.float32)
    dist = jnp.maximum(x2 + c2 - 2.0 * m, 0.0)
    u = 1.0 / (1.0 + dist)
    s = jnp.sum(u, axis=0, keepdims=True)
    o_ref[...] = u * (1.0 / s)


def kernel(x, centers):
    n, d = x.shape
    k = centers.shape[0]
    grid = (pl.cdiv(n, _BLOCK),)
    qt = pl.pallas_call(
        _body,
        grid=grid,
        in_specs=[
            pl.BlockSpec((_BLOCK, d), lambda i: (i, 0)),
            pl.BlockSpec((k, d), lambda i: (0, 0)),
        ],
        out_specs=pl.BlockSpec((k, _BLOCK), lambda i: (0, i)),
        out_shape=jax.ShapeDtypeStruct((k, n), jnp.float32),
    )(x, centers)
    return qt.T


# speedup vs baseline: 1.0631x; 1.0631x over previous
import jax
import jax.numpy as jnp
from jax.experimental import pallas as pl

_BLOCK = 33408


def _body(x_ref, c_ref, o_ref):
    xb = x_ref[...]
    xt = xb.T
    x2 = jnp.sum(xt * xt, axis=0, keepdims=True)
    c = c_ref[...]
    c2 = jnp.sum(c * c, axis=1, keepdims=True)
    m = jax.lax.dot_general(c, xt, (((1,), (0,)), ((), ())),
                            preferred_element_type=jnp.float32)
    dist = jnp.maximum(x2 + c2 - 2.0 * m, 0.0)
    u = 1.0 / (1.0 + dist)
    s = jnp.sum(u, axis=0, keepdims=True)
    o_ref[...] = u * (1.0 / s)


def kernel(x, centers):
    n, d = x.shape
    k = centers.shape[0]
    grid = (pl.cdiv(n, _BLOCK),)
    qt = pl.pallas_call(
        _body,
        grid=grid,
        in_specs=[
            pl.BlockSpec((_BLOCK, d), lambda i: (i, 0)),
            pl.BlockSpec((k, d), lambda i: (0, 0)),
        ],
        out_specs=pl.BlockSpec((k, _BLOCK), lambda i: (0, i)),
        out_shape=jax.ShapeDtypeStruct((k, n), jnp.float32),
    )(x, centers)
    return qt.T


# speedup vs baseline: 1.0929x; 1.0280x over previous
import jax
import jax.numpy as jnp
from jax.experimental import pallas as pl

_BLOCK = 25600


def _body(x_ref, c_ref, o_ref):
    xb = x_ref[...]
    xt = xb.T
    x2 = jnp.sum(xt * xt, axis=0, keepdims=True)
    c = c_ref[...]
    c2 = jnp.sum(c * c, axis=1, keepdims=True)
    m = jax.lax.dot_general(c, xt, (((1,), (0,)), ((), ())),
                            preferred_element_type=jnp.float32)
    dist = jnp.maximum(x2 + c2 - 2.0 * m, 0.0)
    u = 1.0 / (1.0 + dist)
    s = jnp.sum(u, axis=0, keepdims=True)
    o_ref[...] = u * (1.0 / s)


def kernel(x, centers):
    n, d = x.shape
    k = centers.shape[0]
    grid = (pl.cdiv(n, _BLOCK),)
    qt = pl.pallas_call(
        _body,
        grid=grid,
        in_specs=[
            pl.BlockSpec((_BLOCK, d), lambda i: (i, 0)),
            pl.BlockSpec((k, d), lambda i: (0, 0)),
        ],
        out_specs=pl.BlockSpec((k, _BLOCK), lambda i: (0, i)),
        out_shape=jax.ShapeDtypeStruct((k, n), jnp.float32),
    )(x, centers)
    return qt.T
